# Initial kernel scaffold; baseline (speedup 1.0000x reference)
import functools

import jax
import jax.numpy as jnp
from jax import lax
from jax.experimental import pallas as pl
from jax.experimental.pallas import tpu as pltpu
from jax.experimental.pallas import tpu_sc as plsc

NC = 2
NS = 16
NW = NC * NS

E = 320000
D = 128
EPW = E // NW
C = 80
NCHUNK = EPW // C
NG = C // 16


def _dot_body(x_hbm, ei_hbm, out_hbm, idxr, idxc, xr, xc, tmp, outv, sem1, sem2):
    wid = lax.axis_index("s") * NC + lax.axis_index("c")
    wbase = wid * EPW

    lane = lax.iota(jnp.int32, 16)

    def chunk_body(chunk, carry):
        base = pl.multiple_of(wbase + chunk * C, 8)
        pltpu.sync_copy(ei_hbm.at[0, pl.ds(base, C)], idxr)
        pltpu.sync_copy(ei_hbm.at[1, pl.ds(base, C)], idxc)
        cp1 = pltpu.async_copy(x_hbm.at[idxr], xr, sem1)
        cp2 = pltpu.async_copy(x_hbm.at[idxc], xc, sem2)
        cp1.wait()
        cp2.wait()

        def group_body(g, gcarry):
            gb = g * 16
            for e in range(16):
                acc = xr[gb + e, pl.ds(0, 16)] * xc[gb + e, pl.ds(0, 16)]
                for fb in range(1, D // 16):
                    a = xr[gb + e, pl.ds(fb * 16, 16)]
                    b = xc[gb + e, pl.ds(fb * 16, 16)]
                    acc = acc + a * b
                tmp[e, :] = acc
            o = plsc.load_gather(tmp, [lane, jnp.zeros((16,), jnp.int32)])
            for f in range(1, 16):
                o = o + plsc.load_gather(tmp, [lane, jnp.full((16,), f, jnp.int32)])
            outv[pl.ds(gb, 16)] = o
            return gcarry

        lax.fori_loop(0, NG, group_body, 0)
        pltpu.sync_copy(outv, out_hbm.at[pl.ds(base, C)])
        return carry

    lax.fori_loop(0, NCHUNK, chunk_body, 0)


@jax.jit
def _decoder(x, edge_index):
    kfn = functools.partial(
        pl.kernel,
        out_type=jax.ShapeDtypeStruct((E,), jnp.float32),
        mesh=plsc.VectorSubcoreMesh(core_axis_name="c", subcore_axis_name="s"),
        scratch_types=[
            pltpu.VMEM((C,), jnp.int32),
            pltpu.VMEM((C,), jnp.int32),
            pltpu.VMEM((C, D), jnp.float32),
            pltpu.VMEM((C, D), jnp.float32),
            pltpu.VMEM((16, 16), jnp.float32),
            pltpu.VMEM((C,), jnp.float32),
            pltpu.SemaphoreType.DMA,
            pltpu.SemaphoreType.DMA,
        ],
    )(_dot_body)
    return kfn(x, edge_index)


def kernel(x, edge_index):
    return _decoder(x, edge_index)


# speedup vs baseline: 3.5561x; 3.5561x over previous
import functools

import jax
import jax.numpy as jnp
from jax import lax
from jax.experimental import pallas as pl
from jax.experimental.pallas import tpu as pltpu
from jax.experimental.pallas import tpu_sc as plsc

NC = 2
NS = 16
NW = NC * NS

E = 320000
D = 128
EPW = E // NW
C = 80
NCHUNK = EPW // C
NG = C // 16


def _dot_body(x_hbm, ei_hbm, out_hbm, idxr, idxc, xr, xc, tmp, outv, sem1, sem2):
    wid = lax.axis_index("s") * NC + lax.axis_index("c")
    wbase = wid * EPW

    lane = lax.iota(jnp.int32, 16)

    def chunk_body(chunk, carry):
        base = pl.multiple_of(wbase + chunk * C, 8)
        pltpu.sync_copy(ei_hbm.at[pl.ds(base, C)], idxr)
        pltpu.sync_copy(ei_hbm.at[pl.ds(E + base, C)], idxc)
        cp1 = pltpu.async_copy(x_hbm.at[idxr], xr, sem1)
        cp2 = pltpu.async_copy(x_hbm.at[idxc], xc, sem2)
        cp1.wait()
        cp2.wait()

        def group_body(g, gcarry):
            gb = g * 16
            for e in range(16):
                acc = xr[gb + e, pl.ds(0, 16)] * xc[gb + e, pl.ds(0, 16)]
                for fb in range(1, D // 16):
                    a = xr[gb + e, pl.ds(fb * 16, 16)]
                    b = xc[gb + e, pl.ds(fb * 16, 16)]
                    acc = acc + a * b
                tmp[pl.ds(e * 16, 16)] = acc
            lane16 = lane * 16
            o = plsc.load_gather(tmp, [lane16])
            for f in range(1, 16):
                o = o + plsc.load_gather(tmp, [lane16 + f])
            outv[pl.ds(gb, 16)] = o
            return gcarry

        lax.fori_loop(0, NG, group_body, 0)
        pltpu.sync_copy(outv, out_hbm.at[pl.ds(base, C)])
        return carry

    lax.fori_loop(0, NCHUNK, chunk_body, 0)


@jax.jit
def _decoder(x, edge_index):
    kfn = functools.partial(
        pl.kernel,
        out_type=jax.ShapeDtypeStruct((E,), jnp.float32),
        mesh=plsc.VectorSubcoreMesh(core_axis_name="c", subcore_axis_name="s"),
        compiler_params=pltpu.CompilerParams(
            needs_layout_passes=False, use_tc_tiling_on_sc=False
        ),
        scratch_types=[
            pltpu.VMEM((C,), jnp.int32),
            pltpu.VMEM((C,), jnp.int32),
            pltpu.VMEM((C, D), jnp.float32),
            pltpu.VMEM((C, D), jnp.float32),
            pltpu.VMEM((256,), jnp.float32),
            pltpu.VMEM((C,), jnp.float32),
            pltpu.SemaphoreType.DMA,
            pltpu.SemaphoreType.DMA,
        ],
    )(_dot_body)
    return kfn(x, edge_index.reshape(-1))


def kernel(x, edge_index):
    return _decoder(x, edge_index)


# speedup vs baseline: 5.2940x; 1.4887x over previous
import functools

import jax
import jax.numpy as jnp
from jax import lax
from jax.experimental import pallas as pl
from jax.experimental.pallas import tpu as pltpu
from jax.experimental.pallas import tpu_sc as plsc

NC = 2
NS = 16
NW = NC * NS

E = 320000
D = 128
EPW = E // NW
C = 80
NCHUNK = EPW // C
NG = C // 16
assert EPW % C == 0 and C % 16 == 0 and NCHUNK % 2 == 1


def _dot_body(
    x_hbm, ei_hbm, out_hbm,
    idxr0, idxc0, idxr1, idxc1,
    xr0, xc0, xr1, xc1,
    tmp, outv,
    semr0, semc0, semr1, semc1,
):
    wid = lax.axis_index("s") * NC + lax.axis_index("c")
    wbase = wid * EPW

    lane = lax.iota(jnp.int32, 16)
    bufs = ((idxr0, idxc0, xr0, xc0, semr0, semc0),
            (idxr1, idxc1, xr1, xc1, semr1, semc1))

    def issue(g, b):
        idxr, idxc, xr, xc, semr, semc = bufs[b]
        base = pl.multiple_of(wbase + g * C, 8)
        pltpu.sync_copy(ei_hbm.at[pl.ds(base, C)], idxr)
        pltpu.sync_copy(ei_hbm.at[pl.ds(E + base, C)], idxc)
        pltpu.async_copy(x_hbm.at[idxr], xr, semr)
        pltpu.async_copy(x_hbm.at[idxc], xc, semc)

    def wait(b):
        idxr, idxc, xr, xc, semr, semc = bufs[b]
        pltpu.make_async_copy(x_hbm.at[idxr], xr, semr).wait()
        pltpu.make_async_copy(x_hbm.at[idxc], xc, semc).wait()

    def compute(g, b):
        idxr, idxc, xr, xc, semr, semc = bufs[b]
        base = pl.multiple_of(wbase + g * C, 8)

        def group_body(gg, gcarry):
            gb = gg * 16
            for e in range(16):
                acc = xr[gb + e, pl.ds(0, 16)] * xc[gb + e, pl.ds(0, 16)]
                for fb in range(1, D // 16):
                    a = xr[gb + e, pl.ds(fb * 16, 16)]
                    b_ = xc[gb + e, pl.ds(fb * 16, 16)]
                    acc = acc + a * b_
                tmp[pl.ds(e * 16, 16)] = acc
            lane16 = lane * 16
            o = plsc.load_gather(tmp, [lane16])
            for f in range(1, 16):
                o = o + plsc.load_gather(tmp, [lane16 + f])
            outv[pl.ds(gb, 16)] = o
            return gcarry

        lax.fori_loop(0, NG, group_body, 0)
        pltpu.sync_copy(outv, out_hbm.at[pl.ds(base, C)])

    issue(0, 0)

    def chunk_pair(g, carry):
        wait(0)
        issue(g + 1, 1)
        compute(g, 0)
        wait(1)
        issue(g + 2, 0)
        compute(g + 1, 1)
        return carry

    lax.fori_loop(0, NCHUNK // 2, lambda i, c: chunk_pair(i * 2, c), 0)
    wait(0)
    compute(NCHUNK - 1, 0)


@jax.jit
def _decoder(x, edge_index):
    kfn = functools.partial(
        pl.kernel,
        out_type=jax.ShapeDtypeStruct((E,), jnp.float32),
        mesh=plsc.VectorSubcoreMesh(core_axis_name="c", subcore_axis_name="s"),
        compiler_params=pltpu.CompilerParams(
            needs_layout_passes=False, use_tc_tiling_on_sc=False
        ),
        scratch_types=[
            pltpu.VMEM((C,), jnp.int32),
            pltpu.VMEM((C,), jnp.int32),
            pltpu.VMEM((C,), jnp.int32),
            pltpu.VMEM((C,), jnp.int32),
            pltpu.VMEM((C, D), jnp.float32),
            pltpu.VMEM((C, D), jnp.float32),
            pltpu.VMEM((C, D), jnp.float32),
            pltpu.VMEM((C, D), jnp.float32),
            pltpu.VMEM((256,), jnp.float32),
            pltpu.VMEM((C,), jnp.float32),
            pltpu.SemaphoreType.DMA,
            pltpu.SemaphoreType.DMA,
            pltpu.SemaphoreType.DMA,
            pltpu.SemaphoreType.DMA,
        ],
    )(_dot_body)
    return kfn(x, edge_index.reshape(-1))


def kernel(x, edge_index):
    return _decoder(x, edge_index)
